# baseline (speedup 1.0000x reference)
import functools

import jax
import jax.numpy as jnp
from jax import lax
from jax.experimental import pallas as pl
from jax.experimental.pallas import tpu as pltpu
from jax.experimental.pallas import tpu_sc as plsc

NC, NS = 2, 16
NW = NC * NS
V, D = 1000000, 32
NTC = 7813
A_UNITS = 245
B, T = 4096, 200
NTG, NBG = 25, 32
B_UNITS = 25


def _mesh():
    return plsc.VectorSubcoreMesh(
        core_axis_name="c", subcore_axis_name="s",
        num_cores=NC, num_subcores=NS,
    )


@jax.jit
def _run(idx, table):
    table_t = table.T
    idx5 = (idx.T.reshape(NTG, 8, NBG, 128)
            .transpose(0, 2, 1, 3).reshape(NTG, NBG, 1024))

    @functools.partial(
        pl.kernel,
        mesh=_mesh(),
        out_type=jax.ShapeDtypeStruct((V // 4, 128), jnp.float32),
        scratch_types=[
            pltpu.VMEM((32, 128), jnp.float32),
            pltpu.VMEM((32, 128), jnp.float32),
        ],
        compiler_params=pltpu.CompilerParams(
            use_tc_tiling_on_sc=True, disable_bounds_checks=True,
            needs_layout_passes=False,
        ),
    )
    def ka(table_hbm, scratch_hbm, tin, tout):
        wid = lax.axis_index("s") * NC + lax.axis_index("c")
        iota = lax.iota(jnp.int32, 16)

        @pl.loop(0, A_UNITS)
        def _(u):
            tc = wid + u * NW

            @pl.when(tc < NTC)
            def _():
                col0 = pl.multiple_of(tc * 128, 128)
                pltpu.sync_copy(table_hbm.at[:, pl.ds(col0, 128)], tin)

                @pl.loop(0, 32)
                def _(q):
                    for s in range(4):
                        kv = q * 4 + s
                        kvec = jnp.full((16,), kv, jnp.int32)
                        for h in range(2):
                            vals = plsc.load_gather(tin, [iota + 16 * h, kvec])
                            tout[q, pl.ds(s * 32 + 16 * h, 16)] = vals

                row0 = pl.multiple_of(tc * 32, 8)

                @pl.when(tc < NTC - 1)
                def _():
                    pltpu.sync_copy(tout, scratch_hbm.at[pl.ds(row0, 32)])

                @pl.when(tc == NTC - 1)
                def _():
                    pltpu.sync_copy(tout.at[pl.ds(0, 16)],
                                    scratch_hbm.at[pl.ds(row0, 16)])

    scratch = ka(table_t)
    scratch2 = scratch.reshape(V, D)

    @functools.partial(
        pl.kernel,
        mesh=_mesh(),
        out_type=jax.ShapeDtypeStruct((NTG, 8, 4, NBG, 8, 128), jnp.float32),
        scratch_types=[
            pltpu.VMEM((1024,), jnp.int32),
            pltpu.VMEM((1024, D), jnp.float32),
            pltpu.VMEM((8, 4, 8, 128), jnp.float32),
            pltpu.SemaphoreType.DMA,
        ],
        compiler_params=pltpu.CompilerParams(use_tc_tiling_on_sc=False, needs_layout_passes=False),
    )
    def kb(idx_hbm, scratch_hbm, out_hbm, idxv, rows, obuf, sem):
        wid = lax.axis_index("s") * NC + lax.axis_index("c")
        iota32 = lax.iota(jnp.int32, 16) * 32

        @pl.loop(0, B_UNITS)
        def _(u):
            q = wid + u * NW
            tg = q // NBG
            bg = q % NBG
            pltpu.sync_copy(idx_hbm.at[tg, bg], idxv)
            pltpu.async_copy(scratch_hbm.at[idxv], rows, sem).wait()

            @pl.loop(0, 8)
            def _(r):
                for tr in range(4):
                    for rr in range(8):
                        c = 8 * tr + rr
                        cvec = jnp.full((16,), c, jnp.int32)
                        for m in range(8):
                            jvec = jnp.full((16,), r * 128 + 16 * m,
                                            jnp.int32) + lax.iota(jnp.int32, 16)
                            vals = plsc.load_gather(rows, [jvec, cvec])
                            obuf[r, tr, rr, pl.ds(16 * m, 16)] = vals
                for tr in range(4):
                    pltpu.sync_copy(obuf.at[r, tr], out_hbm.at[tg, r, tr, bg])

    out5 = kb(idx5, scratch2)
    out = (out5.transpose(3, 5, 0, 1, 2, 4).reshape(B, T, D))
    return out


def kernel(idx, token_embedding_table):
    return _run(idx.astype(jnp.int32), token_embedding_table)


# speedup vs baseline: 1.4223x; 1.4223x over previous
import functools

import jax
import jax.numpy as jnp
from jax import lax
from jax.experimental import pallas as pl
from jax.experimental.pallas import tpu as pltpu
from jax.experimental.pallas import tpu_sc as plsc

NC, NS = 2, 16
NW = NC * NS
V, D = 1000000, 32
B, T = 4096, 200
NTG, NBG = 25, 32
B_UNITS = 25


def _mesh():
    return plsc.VectorSubcoreMesh(
        core_axis_name="c", subcore_axis_name="s",
        num_cores=NC, num_subcores=NS,
    )


@jax.jit
def _run(idx, table):
    idx5 = (idx.T.reshape(NTG, 8, NBG, 128)
            .transpose(0, 2, 1, 3).reshape(NTG, NBG, 1024))

    @functools.partial(
        pl.kernel,
        mesh=_mesh(),
        out_type=jax.ShapeDtypeStruct((NTG, 8, 4, NBG, 8, 128), jnp.float32),
        scratch_types=[
            pltpu.VMEM((1024,), jnp.int32),
            pltpu.VMEM((1024, D), jnp.float32),
            pltpu.VMEM((8, 4, 8, 128), jnp.float32),
            pltpu.SemaphoreType.DMA,
        ],
        compiler_params=pltpu.CompilerParams(
            use_tc_tiling_on_sc=False, needs_layout_passes=False),
    )
    def kb(idx_hbm, table_hbm, out_hbm, idxv, rows, obuf, sem):
        wid = lax.axis_index("s") * NC + lax.axis_index("c")

        @pl.loop(0, B_UNITS)
        def _(u):
            q = wid + u * NW
            tg = q // NBG
            bg = q % NBG
            pltpu.sync_copy(idx_hbm.at[tg, bg], idxv)
            pltpu.async_copy(table_hbm.at[idxv], rows, sem).wait()

            @pl.loop(0, 8)
            def _(r):
                for tr in range(4):
                    for rr in range(8):
                        c = 8 * tr + rr
                        cvec = jnp.full((16,), c, jnp.int32)
                        for m in range(8):
                            jvec = jnp.full((16,), r * 128 + 16 * m,
                                            jnp.int32) + lax.iota(jnp.int32, 16)
                            vals = plsc.load_gather(rows, [jvec, cvec])
                            obuf[r, tr, rr, pl.ds(16 * m, 16)] = vals
                for tr in range(4):
                    pltpu.sync_copy(obuf.at[r, tr], out_hbm.at[tg, r, tr, bg])

    out5 = kb(idx5, table)
    out = (out5.transpose(3, 5, 0, 1, 2, 4).reshape(B, T, D))
    return out


def kernel(idx, token_embedding_table):
    return _run(idx.astype(jnp.int32), token_embedding_table)
